# Initial kernel scaffold; baseline (speedup 1.0000x reference)
import jax
import jax.numpy as jnp
from jax.experimental import pallas as pl


def kernel(states, mask, u_int, u_float):
    raise NotImplementedError("write your pallas kernel here")


# speedup vs baseline: 3.4153x; 3.4153x over previous
import functools

import jax
import jax.numpy as jnp
from jax import lax
from jax.experimental import pallas as pl
from jax.experimental.pallas import tpu as pltpu
from jax.experimental.pallas import tpu_sc as plsc

B = 4096
NV = 1000
NVP = 1024
W = NVP // 4
ACT = 8
NW = 32
RPW = B // NW
G = RPW // 16
NFLT = RPW * ACT


def _body(words_hbm, u_hbm, uf_hbm, ia_hbm, fa_hbm, words_v, u_v, uf_v, ia_v, fa_v):
    wid = lax.axis_index("s") * 2 + lax.axis_index("c")
    base = wid * RPW

    pltpu.sync_copy(words_hbm.at[wid], words_v)
    pltpu.sync_copy(u_hbm.at[pl.ds(base, RPW)], u_v)
    pltpu.sync_copy(uf_hbm.at[pl.ds(base * ACT, NFLT)], uf_v)

    rep = jnp.full((16,), 0x01010101, jnp.int32)
    bias = jnp.full((16,), 0x08080808, jnp.int32)
    four = jnp.full((16,), 4, jnp.int32)
    zero = jnp.zeros((16,), jnp.int32)

    for g in range(G):
        col = g * 16

        def p1(w, tot):
            v = words_v[w, pl.ds(col, 16)]
            return tot + lax.shift_right_logical(v * rep, 24)

        tot = lax.fori_loop(0, W, p1, zero)
        u = u_v[pl.ds(col, 16)]
        action = (u * tot.astype(jnp.float32)).astype(jnp.int32)

        def p2(w, carry):
            r, cnt = carry
            v = words_v[w, pl.ds(col, 16)]
            t_word = v * rep
            p = lax.shift_right_logical(t_word, 24)
            t = action - r
            tt = jnp.minimum(t, four)
            d = tt * rep + bias - t_word
            e = lax.shift_right_logical(d, 3) & rep
            c4 = lax.shift_right_logical(e * rep, 24)
            c4 = jnp.where(t < zero, zero, c4)
            return (r + p, cnt + c4)

        _, cnt = lax.fori_loop(0, W, p2, (zero, zero))
        ia_v[pl.ds(col, 16)] = cnt

    def pf(i, _):
        x = uf_v[pl.ds(i * 16, 16)]
        fa_v[pl.ds(i * 16, 16)] = x * 2.0 - 1.0
        return 0

    lax.fori_loop(0, NFLT // 16, pf, 0)

    pltpu.sync_copy(ia_v, ia_hbm.at[pl.ds(base, RPW)])
    pltpu.sync_copy(fa_v, fa_hbm.at[pl.ds(base * ACT, NFLT)])


_sc_call = pl.kernel(
    _body,
    out_type=(
        jax.ShapeDtypeStruct((B,), jnp.int32),
        jax.ShapeDtypeStruct((B * ACT,), jnp.float32),
    ),
    mesh=plsc.VectorSubcoreMesh(core_axis_name="c", subcore_axis_name="s"),
    scratch_types=[
        pltpu.VMEM((W, RPW), jnp.int32),
        pltpu.VMEM((RPW,), jnp.float32),
        pltpu.VMEM((NFLT,), jnp.float32),
        pltpu.VMEM((RPW,), jnp.int32),
        pltpu.VMEM((NFLT,), jnp.float32),
    ],
)


@jax.jit
def kernel(states, mask, u_int, u_float):
    del states
    m = jnp.pad(mask.astype(jnp.uint8), ((0, 0), (0, NVP - NV)))
    m4 = m.reshape(B, W, 4).astype(jnp.int32)
    words = m4[:, :, 0] | (m4[:, :, 1] << 8) | (m4[:, :, 2] << 16) | (m4[:, :, 3] << 24)
    slabs = words.reshape(NW, RPW, W).transpose(0, 2, 1)
    ia, fa = _sc_call(slabs, u_int, u_float.reshape(B * ACT))
    return ia, fa.reshape(B, ACT)


# speedup vs baseline: 5.5706x; 1.6311x over previous
import jax
import jax.numpy as jnp
from jax import lax
from jax.experimental import pallas as pl
from jax.experimental.pallas import tpu as pltpu
from jax.experimental.pallas import tpu_sc as plsc

B = 4096
NV = 1000
W = NV // 4
ACT = 8
NW = 32
RPW = B // NW
G = RPW // 16


def _body(words_hbm, u_hbm, uf_hbm, ia_hbm, fa_hbm, words_v, u_v, uf_v, ia_v, fa_v):
    wid = lax.axis_index("s") * 2 + lax.axis_index("c")
    base = wid * RPW

    pltpu.sync_copy(words_hbm.at[pl.ds(base * W, RPW * W)], words_v)
    pltpu.sync_copy(u_hbm.at[pl.ds(base, RPW)], u_v)
    pltpu.sync_copy(uf_hbm.at[pl.ds(base * ACT, RPW * ACT)], uf_v)

    iota = lax.iota(jnp.int32, 16)
    rep = jnp.full((16,), 0x01010101, jnp.int32)
    bias = jnp.full((16,), 0x08080808, jnp.int32)
    four = jnp.full((16,), 4, jnp.int32)
    zero = jnp.zeros((16,), jnp.int32)
    ones = jnp.full((16,), 1, jnp.int32)

    for g in range(G):
        rows = (iota + (g * 16)) * W

        def p1(w, carry):
            idx, tot = carry
            v = plsc.load_gather(words_v, [idx])
            return (idx + ones, tot + lax.shift_right_logical(v * rep, 24))

        _, tot = lax.fori_loop(0, W, p1, (rows, zero))
        u = u_v[pl.ds(g * 16, 16)]
        action = (u * tot.astype(jnp.float32)).astype(jnp.int32)

        def p2(w, carry):
            idx, r, cnt = carry
            v = plsc.load_gather(words_v, [idx])
            t_word = v * rep
            p = lax.shift_right_logical(t_word, 24)
            t = action - r
            tt = jnp.minimum(t, four)
            d = tt * rep + bias - t_word
            e = lax.shift_right_logical(d, 3) & rep
            c4 = lax.shift_right_logical(e * rep, 24)
            c4 = jnp.where(t < zero, zero, c4)
            return (idx + ones, r + p, cnt + c4)

        _, _, cnt = lax.fori_loop(0, W, p2, (rows, zero, zero))
        ia_v[pl.ds(g * 16, 16)] = cnt

    def pf(k, _):
        x = uf_v[pl.ds(k * 16, 16)]
        fa_v[pl.ds(k * 16, 16)] = x * 2.0 - 1.0
        return 0

    lax.fori_loop(0, RPW * ACT // 16, pf, 0)

    pltpu.sync_copy(ia_v, ia_hbm.at[pl.ds(base, RPW)])
    pltpu.sync_copy(fa_v, fa_hbm.at[pl.ds(base * ACT, RPW * ACT)])


_sc_call = pl.kernel(
    _body,
    out_type=(
        jax.ShapeDtypeStruct((B,), jnp.int32),
        jax.ShapeDtypeStruct((B * ACT,), jnp.float32),
    ),
    mesh=plsc.VectorSubcoreMesh(core_axis_name="c", subcore_axis_name="s"),
    compiler_params=pltpu.CompilerParams(needs_layout_passes=False),
    scratch_types=[
        pltpu.VMEM((RPW * W,), jnp.int32),
        pltpu.VMEM((RPW,), jnp.float32),
        pltpu.VMEM((RPW * ACT,), jnp.float32),
        pltpu.VMEM((RPW,), jnp.int32),
        pltpu.VMEM((RPW * ACT,), jnp.float32),
    ],
)


@jax.jit
def kernel(states, mask, u_int, u_float):
    del states
    words = lax.bitcast_convert_type(
        mask.astype(jnp.uint8).reshape(B, W, 4), jnp.int32
    ).reshape(B * W)
    ia, fa = _sc_call(words, u_int, u_float.reshape(B * ACT))
    return ia, fa.reshape(B, ACT)
